# baseline (speedup 1.0000x reference)
import functools

import jax
import jax.numpy as jnp
from jax import lax
from jax.experimental import pallas as pl
from jax.experimental.pallas import tpu as pltpu
from jax.experimental.pallas import tpu_sc as plsc

N_NODES = 10000
D_FEAT = 128
E_EDGES = 320000

NC, NS = 2, 16
NW = NC * NS
N_PAD = 10240
ROWS_PT = N_PAD // NS
BB = 128
E_PT = E_EDGES // NW
E_PAD_PT = 10240
NBLK_PT = E_PAD_PT // BB
KSB = 20
NSB = NBLK_PT // KSB

_sc_mesh = plsc.VectorSubcoreMesh(core_axis_name="c", subcore_axis_name="s")


@functools.partial(
    pl.kernel,
    mesh=_sc_mesh,
    out_type=(
        jax.ShapeDtypeStruct((NC, N_PAD, D_FEAT), jnp.float32),
        jax.ShapeDtypeStruct((NC, N_PAD), jnp.float32),
    ),
    scratch_types=[
        pltpu.VMEM((KSB, BB), jnp.int32),
        pltpu.VMEM((KSB, BB), jnp.int32),
        pltpu.VMEM((KSB, BB), jnp.int32),
        pltpu.VMEM((KSB, BB), jnp.int32),
        pltpu.VMEM((BB, D_FEAT), jnp.float32),
        pltpu.VMEM((BB, D_FEAT), jnp.float32),
        pltpu.VMEM((BB,), jnp.float32),
        pltpu.VMEM_SHARED((N_PAD, D_FEAT), jnp.float32),
        pltpu.VMEM_SHARED((N_PAD,), jnp.float32),
        pltpu.SemaphoreType.DMA,
        pltpu.SemaphoreType.DMA,
        pltpu.SemaphoreType.DMA,
    ],
)
def _scatter_sum_sc(emb_hbm, src_hbm, dst_hbm, zrow_hbm, zdeg_hbm,
                    sums_out, deg_out,
                    srcA_v, srcB_v, dstA_v, dstB_v, rows0_v, rows1_v, ones_v,
                    acc_sh, deg_sh, sem0, sem1, semi):
    c = lax.axis_index("c")
    s = lax.axis_index("s")
    wid = s * NC + c
    bufs = ((rows0_v, sem0), (rows1_v, sem1))
    sbufs = ((srcA_v, dstA_v), (srcB_v, dstB_v))

    for i in range(BB // 16):
        ones_v[pl.ds(i * 16, 16)] = jnp.ones((16,), jnp.float32)

    pltpu.sync_copy(zrow_hbm, acc_sh.at[pl.ds(s * ROWS_PT, ROWS_PT)])
    pltpu.sync_copy(zdeg_hbm, deg_sh.at[pl.ds(s * ROWS_PT, ROWS_PT)])

    pltpu.sync_copy(src_hbm.at[wid, 0], srcA_v)
    pltpu.sync_copy(dst_hbm.at[wid, 0], dstA_v)

    plsc.subcore_barrier()

    for q in range(NSB):
        src_v, dst_v = sbufs[q % 2]
        src_n, dst_n = sbufs[(q + 1) % 2]
        idx_cps = ()
        if q + 1 < NSB:
            idx_cps = (
                pltpu.async_copy(src_hbm.at[wid, q + 1], src_n, semi),
                pltpu.async_copy(dst_hbm.at[wid, q + 1], dst_n, semi),
            )

        def blk(j, carry):
            pltpu.async_copy(emb_hbm.at[src_v.at[j]], rows0_v, sem0).wait()
            pltpu.sync_copy(rows0_v, acc_sh.at[dst_v.at[j]], add=True)
            pltpu.sync_copy(ones_v, deg_sh.at[dst_v.at[j]], add=True)
            return carry

        lax.fori_loop(0, KSB, blk, 0)

        for cp in idx_cps:
            cp.wait()

    plsc.subcore_barrier()

    pltpu.sync_copy(acc_sh.at[pl.ds(s * ROWS_PT, ROWS_PT)],
                    sums_out.at[c, pl.ds(s * ROWS_PT, ROWS_PT)])
    pltpu.sync_copy(deg_sh.at[pl.ds(s * ROWS_PT, ROWS_PT)],
                    deg_out.at[c, pl.ds(s * ROWS_PT, ROWS_PT)])


def _combine_body(sums_ref, deg_ref, out_ref):
    t = sums_ref[0] + sums_ref[1]
    d = deg_ref[0] + deg_ref[1]
    dcol = d[:, None]
    out_ref[...] = jnp.where(dcol > 0, t / jnp.maximum(dcol, 1.0),
                             jnp.zeros_like(t))


_ROWS_BLK = 1024
_combine = pl.pallas_call(
    _combine_body,
    grid=(N_PAD // _ROWS_BLK,),
    in_specs=[
        pl.BlockSpec((NC, _ROWS_BLK, D_FEAT), lambda i: (0, i, 0)),
        pl.BlockSpec((NC, _ROWS_BLK), lambda i: (0, i)),
    ],
    out_specs=pl.BlockSpec((_ROWS_BLK, D_FEAT), lambda i: (i, 0)),
    out_shape=jax.ShapeDtypeStruct((N_NODES, D_FEAT), jnp.float32),
)


def kernel(embeddings, edge_index):
    src = edge_index[0].astype(jnp.int32).reshape(NW, E_PT)
    dst = edge_index[1].astype(jnp.int32).reshape(NW, E_PT)
    pad_e = E_PAD_PT - E_PT
    src = jnp.pad(src, ((0, 0), (0, pad_e)))
    padvec = N_NODES + (jnp.arange(pad_e, dtype=jnp.int32)
                        % (N_PAD - N_NODES))
    dst = jnp.concatenate(
        [dst, jnp.broadcast_to(padvec, (NW, pad_e))], axis=1)
    src = src.reshape(NW, NSB, KSB, BB)
    dst = dst.reshape(NW, NSB, KSB, BB)
    zrow = jnp.zeros((ROWS_PT, D_FEAT), jnp.float32)
    zdeg = jnp.zeros((ROWS_PT,), jnp.float32)
    sums, deg = _scatter_sum_sc(embeddings, src, dst, zrow, zdeg)
    return _combine(sums, deg)


# speedup vs baseline: 2.4667x; 2.4667x over previous
import functools

import jax
import jax.numpy as jnp
from jax import lax
from jax.experimental import pallas as pl
from jax.experimental.pallas import tpu as pltpu
from jax.experimental.pallas import tpu_sc as plsc

N_NODES = 10000
D_FEAT = 128
E_EDGES = 320000

NC, NS = 2, 16
NW = NC * NS
N_PAD = 10240
ROWS_PT = N_PAD // NS
BB = 80
E_PT = E_EDGES // NW
NBLK_PT = E_PT // BB
KSB = 25
NSB = NBLK_PT // KSB

_sc_mesh = plsc.VectorSubcoreMesh(core_axis_name="c", subcore_axis_name="s")


@functools.partial(
    pl.kernel,
    mesh=_sc_mesh,
    out_type=(
        jax.ShapeDtypeStruct((NC, N_PAD, D_FEAT), jnp.float32),
        jax.ShapeDtypeStruct((NC, N_PAD), jnp.float32),
    ),
    scratch_types=[
        pltpu.VMEM((KSB, BB), jnp.int32),
        pltpu.VMEM((KSB, BB), jnp.int32),
        pltpu.VMEM((KSB, BB), jnp.int32),
        pltpu.VMEM((KSB, BB), jnp.int32),
        pltpu.VMEM((BB, D_FEAT), jnp.float32),
        pltpu.VMEM((BB, D_FEAT), jnp.float32),
        pltpu.VMEM((BB,), jnp.float32),
        pltpu.VMEM_SHARED((N_PAD, D_FEAT), jnp.float32),
        pltpu.VMEM_SHARED((N_PAD,), jnp.float32),
        pltpu.SemaphoreType.DMA,
        pltpu.SemaphoreType.DMA,
        pltpu.SemaphoreType.DMA,
        pltpu.SemaphoreType.DMA,
        pltpu.SemaphoreType.DMA,
        pltpu.SemaphoreType.DMA,
    ],
)
def _scatter_sum_sc(emb_hbm, src_hbm, dst_hbm, zrow_hbm, zdeg_hbm,
                    sums_out, deg_out,
                    srcA_v, srcB_v, dstA_v, dstB_v, rows0_v, rows1_v, ones_v,
                    acc_sh, deg_sh, sem0, sem1, ssem0, ssem1, osem, semi):
    c = lax.axis_index("c")
    s = lax.axis_index("s")
    wid = s * NC + c
    bufs = ((rows0_v, sem0, ssem0), (rows1_v, sem1, ssem1))
    sbufs = ((srcA_v, dstA_v), (srcB_v, dstB_v))

    for i in range(BB // 16):
        ones_v[pl.ds(i * 16, 16)] = jnp.ones((16,), jnp.float32)

    pltpu.sync_copy(zrow_hbm, acc_sh.at[pl.ds(s * ROWS_PT, ROWS_PT)])
    pltpu.sync_copy(zdeg_hbm, deg_sh.at[pl.ds(s * ROWS_PT, ROWS_PT)])

    pltpu.sync_copy(src_hbm.at[wid, 0], srcA_v)
    pltpu.sync_copy(dst_hbm.at[wid, 0], dstA_v)

    plsc.subcore_barrier()

    for q in range(NSB):
        src_v, dst_v = sbufs[q % 2]
        src_n, dst_n = sbufs[(q + 1) % 2]
        idx_cps = ()
        if q + 1 < NSB:
            idx_cps = (
                pltpu.async_copy(src_hbm.at[wid, q + 1], src_n, semi),
                pltpu.async_copy(dst_hbm.at[wid, q + 1], dst_n, semi),
            )

        gcp = [None, None]
        scp = [None, None]
        ocp = []
        gcp[0] = pltpu.async_copy(emb_hbm.at[src_v.at[0]], rows0_v, sem0)
        for j in range(KSB):
            b = j % 2
            rows_v, _, ssem = bufs[b]
            gcp[b].wait()
            scp[b] = pltpu.async_copy(rows_v, acc_sh.at[dst_v.at[j]],
                                      ssem, add=True)
            ocp.append(pltpu.async_copy(ones_v, deg_sh.at[dst_v.at[j]],
                                        osem, add=True))
            nb = 1 - b
            if j + 1 < KSB:
                if scp[nb] is not None:
                    scp[nb].wait()
                gcp[nb] = pltpu.async_copy(emb_hbm.at[src_v.at[j + 1]],
                                           bufs[nb][0], bufs[nb][1])

        for d in scp:
            if d is not None:
                d.wait()
        for d in ocp:
            d.wait()
        for cp in idx_cps:
            cp.wait()

    plsc.subcore_barrier()

    pltpu.sync_copy(acc_sh.at[pl.ds(s * ROWS_PT, ROWS_PT)],
                    sums_out.at[c, pl.ds(s * ROWS_PT, ROWS_PT)])
    pltpu.sync_copy(deg_sh.at[pl.ds(s * ROWS_PT, ROWS_PT)],
                    deg_out.at[c, pl.ds(s * ROWS_PT, ROWS_PT)])


def _combine_body(sums_ref, deg_ref, out_ref):
    t = sums_ref[0] + sums_ref[1]
    d = deg_ref[0] + deg_ref[1]
    dcol = d[:, None]
    out_ref[...] = jnp.where(dcol > 0, t / jnp.maximum(dcol, 1.0),
                             jnp.zeros_like(t))


_ROWS_BLK = 1024
_combine = pl.pallas_call(
    _combine_body,
    grid=(N_PAD // _ROWS_BLK,),
    in_specs=[
        pl.BlockSpec((NC, _ROWS_BLK, D_FEAT), lambda i: (0, i, 0)),
        pl.BlockSpec((NC, _ROWS_BLK), lambda i: (0, i)),
    ],
    out_specs=pl.BlockSpec((_ROWS_BLK, D_FEAT), lambda i: (i, 0)),
    out_shape=jax.ShapeDtypeStruct((N_NODES, D_FEAT), jnp.float32),
)


def kernel(embeddings, edge_index):
    src = edge_index[0].astype(jnp.int32).reshape(NW, NSB, KSB, BB)
    dst = edge_index[1].astype(jnp.int32).reshape(NW, NSB, KSB, BB)
    zrow = jnp.zeros((ROWS_PT, D_FEAT), jnp.float32)
    zdeg = jnp.zeros((ROWS_PT,), jnp.float32)
    sums, deg = _scatter_sum_sc(embeddings, src, dst, zrow, zdeg)
    return _combine(sums, deg)


# speedup vs baseline: 2.4717x; 1.0020x over previous
import functools

import jax
import jax.numpy as jnp
from jax import lax
from jax.experimental import pallas as pl
from jax.experimental.pallas import tpu as pltpu
from jax.experimental.pallas import tpu_sc as plsc

N_NODES = 10000
D_FEAT = 128
E_EDGES = 320000

NC, NS = 2, 16
NW = NC * NS
N_PAD = 10240
ROWS_PT = N_PAD // NS
BB = 80
E_PT = E_EDGES // NW
NBLK_PT = E_PT // BB
KSB = 25
NSB = NBLK_PT // KSB

_sc_mesh = plsc.VectorSubcoreMesh(core_axis_name="c", subcore_axis_name="s")


@functools.partial(
    pl.kernel,
    mesh=_sc_mesh,
    out_type=(
        jax.ShapeDtypeStruct((NC, N_PAD, D_FEAT), jnp.float32),
        jax.ShapeDtypeStruct((NC, N_PAD), jnp.float32),
    ),
    scratch_types=[
        pltpu.VMEM((KSB, BB), jnp.int32),
        pltpu.VMEM((KSB, BB), jnp.int32),
        pltpu.VMEM((KSB, BB), jnp.int32),
        pltpu.VMEM((KSB, BB), jnp.int32),
        pltpu.VMEM((BB, D_FEAT), jnp.float32),
        pltpu.VMEM((BB, D_FEAT), jnp.float32),
        pltpu.VMEM((BB, D_FEAT), jnp.float32),
        pltpu.VMEM((BB,), jnp.float32),
        pltpu.VMEM_SHARED((N_PAD, D_FEAT), jnp.float32),
        pltpu.VMEM_SHARED((N_PAD,), jnp.float32),
        pltpu.SemaphoreType.DMA,
        pltpu.SemaphoreType.DMA,
        pltpu.SemaphoreType.DMA,
        pltpu.SemaphoreType.DMA,
        pltpu.SemaphoreType.DMA,
        pltpu.SemaphoreType.DMA,
        pltpu.SemaphoreType.DMA,
        pltpu.SemaphoreType.DMA,
    ],
)
def _scatter_sum_sc(emb_hbm, src_hbm, dst_hbm, zrow_hbm, zdeg_hbm,
                    sums_out, deg_out,
                    srcA_v, srcB_v, dstA_v, dstB_v,
                    rows0_v, rows1_v, rows2_v, ones_v,
                    acc_sh, deg_sh, sem0, sem1, sem2,
                    ssem0, ssem1, ssem2, osem, semi):
    c = lax.axis_index("c")
    s = lax.axis_index("s")
    wid = s * NC + c
    bufs = ((rows0_v, sem0, ssem0), (rows1_v, sem1, ssem1),
            (rows2_v, sem2, ssem2))
    nbuf = len(bufs)
    sbufs = ((srcA_v, dstA_v), (srcB_v, dstB_v))

    for i in range(BB // 16):
        ones_v[pl.ds(i * 16, 16)] = jnp.ones((16,), jnp.float32)

    pltpu.sync_copy(zrow_hbm, acc_sh.at[pl.ds(s * ROWS_PT, ROWS_PT)])
    pltpu.sync_copy(zdeg_hbm, deg_sh.at[pl.ds(s * ROWS_PT, ROWS_PT)])

    pltpu.sync_copy(src_hbm.at[wid, 0], srcA_v)
    pltpu.sync_copy(dst_hbm.at[wid, 0], dstA_v)

    plsc.subcore_barrier()

    for q in range(NSB):
        src_v, dst_v = sbufs[q % 2]
        src_n, dst_n = sbufs[(q + 1) % 2]
        idx_cps = ()
        if q + 1 < NSB:
            idx_cps = (
                pltpu.async_copy(src_hbm.at[wid, q + 1], src_n, semi),
                pltpu.async_copy(dst_hbm.at[wid, q + 1], dst_n, semi),
            )

        gcp = [None] * nbuf
        scp = [None] * nbuf
        ocp = []
        gcp[0] = pltpu.async_copy(emb_hbm.at[src_v.at[0]], rows0_v, sem0)
        for j in range(KSB):
            b = j % nbuf
            rows_v, _, ssem = bufs[b]
            gcp[b].wait()
            scp[b] = pltpu.async_copy(rows_v, acc_sh.at[dst_v.at[j]],
                                      ssem, add=True)
            ocp.append(pltpu.async_copy(ones_v, deg_sh.at[dst_v.at[j]],
                                        osem, add=True))
            nb = (j + 1) % nbuf
            if j + 1 < KSB:
                if scp[nb] is not None:
                    scp[nb].wait()
                gcp[nb] = pltpu.async_copy(emb_hbm.at[src_v.at[j + 1]],
                                           bufs[nb][0], bufs[nb][1])

        for d in scp:
            if d is not None:
                d.wait()
        for d in ocp:
            d.wait()
        for cp in idx_cps:
            cp.wait()

    plsc.subcore_barrier()

    pltpu.sync_copy(acc_sh.at[pl.ds(s * ROWS_PT, ROWS_PT)],
                    sums_out.at[c, pl.ds(s * ROWS_PT, ROWS_PT)])
    pltpu.sync_copy(deg_sh.at[pl.ds(s * ROWS_PT, ROWS_PT)],
                    deg_out.at[c, pl.ds(s * ROWS_PT, ROWS_PT)])


def _combine_body(sums_ref, deg_ref, out_ref):
    t = sums_ref[0] + sums_ref[1]
    d = deg_ref[0] + deg_ref[1]
    dcol = d[:, None]
    out_ref[...] = jnp.where(dcol > 0, t / jnp.maximum(dcol, 1.0),
                             jnp.zeros_like(t))


_ROWS_BLK = 1024
_combine = pl.pallas_call(
    _combine_body,
    grid=(N_PAD // _ROWS_BLK,),
    in_specs=[
        pl.BlockSpec((NC, _ROWS_BLK, D_FEAT), lambda i: (0, i, 0)),
        pl.BlockSpec((NC, _ROWS_BLK), lambda i: (0, i)),
    ],
    out_specs=pl.BlockSpec((_ROWS_BLK, D_FEAT), lambda i: (i, 0)),
    out_shape=jax.ShapeDtypeStruct((N_NODES, D_FEAT), jnp.float32),
)


def kernel(embeddings, edge_index):
    src = edge_index[0].astype(jnp.int32).reshape(NW, NSB, KSB, BB)
    dst = edge_index[1].astype(jnp.int32).reshape(NW, NSB, KSB, BB)
    zrow = jnp.zeros((ROWS_PT, D_FEAT), jnp.float32)
    zdeg = jnp.zeros((ROWS_PT,), jnp.float32)
    sums, deg = _scatter_sum_sc(embeddings, src, dst, zrow, zdeg)
    return _combine(sums, deg)
